# baseline (speedup 1.0000x reference)
import functools

import jax
import jax.numpy as jnp
from jax import lax
from jax.experimental import pallas as pl
from jax.experimental.pallas import tpu as pltpu
import jax.experimental.pallas.tpu_sc as plsc

D = 768
DD = 3072
N_EXP = 64
K = 128
BS = 32768
LN_EPS = 1e-5

TOK_BLK = 2048
L = 16
SEG = 2048
NSEG = BS // SEG
EBLK = 8


def _gate_body(x_ref, wg_ref, out_ref, xc_ref):
    logits = lax.dot_general(
        wg_ref[...], x_ref[...],
        dimension_numbers=(((1,), (1,)), ((), ())),
        preferred_element_type=jnp.float32,
    )
    m = jnp.max(logits, axis=0, keepdims=True)
    e = jnp.exp(logits - m)
    out_ref[...] = e / jnp.sum(e, axis=0, keepdims=True)
    xc_ref[...] = x_ref[...]


def _gate_scores_t(xf, Wg):
    return pl.pallas_call(
        _gate_body,
        grid=(BS // TOK_BLK,),
        in_specs=[
            pl.BlockSpec((TOK_BLK, D), lambda i: (i, 0)),
            pl.BlockSpec((N_EXP, D), lambda i: (0, 0)),
        ],
        out_specs=[
            pl.BlockSpec((N_EXP, TOK_BLK), lambda i: (0, i)),
            pl.BlockSpec((TOK_BLK, D), lambda i: (i, 0)),
        ],
        out_shape=[
            jax.ShapeDtypeStruct((N_EXP, BS), jnp.float32),
            jax.ShapeDtypeStruct((BS, D), jnp.float32),
        ],
    )(xf, Wg)


def _thr_body(s_ref, thr_ref, quo_ref):
    bits = lax.bitcast_convert_type(s_ref[...], jnp.int32)

    def step(_, lohi):
        lo, hi = lohi
        mid = (lo + hi + 1) >> 1
        cnt = jnp.sum((bits >= mid).astype(jnp.int32), axis=1, keepdims=True)
        ge = cnt >= K
        return jnp.where(ge, mid, lo), jnp.where(ge, hi, mid - 1)

    lo0 = jnp.zeros((N_EXP, 1), jnp.int32)
    hi0 = jnp.full((N_EXP, 1), 0x3F800000, jnp.int32)
    lo, _ = lax.fori_loop(0, 31, step, (lo0, hi0))
    cnt_gt = jnp.sum((bits > lo).astype(jnp.int32), axis=1, keepdims=True)
    thr_ref[...] = jnp.broadcast_to(
        lax.bitcast_convert_type(lo, jnp.float32), (N_EXP, L))
    quo_ref[...] = jnp.broadcast_to(K - cnt_gt, (N_EXP, L))


def _thresholds(scores_t):
    return pl.pallas_call(
        _thr_body,
        in_specs=[pl.BlockSpec((N_EXP, BS), lambda: (0, 0))],
        out_specs=[
            pl.BlockSpec((N_EXP, L), lambda: (0, 0)),
            pl.BlockSpec((N_EXP, L), lambda: (0, 0)),
        ],
        out_shape=[
            jax.ShapeDtypeStruct((N_EXP, L), jnp.float32),
            jax.ShapeDtypeStruct((N_EXP, L), jnp.int32),
        ],
    )(scores_t)


def _prefix_body(s_ref, thr_ref, quo_ref, p_ref, tri_ref, tri16_ref):
    @pl.when(pl.program_id(0) == 0)
    def _():
        r = lax.broadcasted_iota(jnp.int32, (SEG, SEG), 0)
        c = lax.broadcasted_iota(jnp.int32, (SEG, SEG), 1)
        tri_ref[...] = jnp.where(r <= c, 1.0, 0.0).astype(jnp.bfloat16)
        r16 = lax.broadcasted_iota(jnp.int32, (L, L), 0)
        c16 = lax.broadcasted_iota(jnp.int32, (L, L), 1)
        tri16_ref[...] = jnp.where(r16 < c16, 1.0, 0.0).astype(jnp.bfloat16)

    s = s_ref[...]
    thr = thr_ref[:, :1]
    quo = quo_ref[:, :1].astype(jnp.float32)

    def prefix(mask_f):
        m = mask_f.reshape(EBLK * NSEG, SEG).astype(jnp.bfloat16)
        pref = lax.dot_general(
            m, tri_ref[...], dimension_numbers=(((1,), (0,)), ((), ())),
            preferred_element_type=jnp.float32,
        )
        seg_tot = pref[:, SEG - 1].reshape(EBLK, NSEG).astype(jnp.bfloat16)
        seg_off = lax.dot_general(
            seg_tot, tri16_ref[...], dimension_numbers=(((1,), (0,)), ((), ())),
            preferred_element_type=jnp.float32,
        )
        return (pref.reshape(EBLK, NSEG, SEG)
                + seg_off[:, :, None]).reshape(EBLK, BS)

    p_gt = prefix(jnp.where(s > thr, 1.0, 0.0))
    p_eq = prefix(jnp.where(s == thr, 1.0, 0.0))
    p_ref[...] = (p_gt + jnp.minimum(p_eq, quo)).astype(jnp.int32)


def _sel_prefix(scores_t, thr, quo):
    return pl.pallas_call(
        _prefix_body,
        grid=(N_EXP // EBLK,),
        in_specs=[
            pl.BlockSpec((EBLK, BS), lambda i: (i, 0)),
            pl.BlockSpec((EBLK, L), lambda i: (i, 0)),
            pl.BlockSpec((EBLK, L), lambda i: (i, 0)),
        ],
        out_specs=pl.BlockSpec((EBLK, BS), lambda i: (i, 0)),
        out_shape=jax.ShapeDtypeStruct((N_EXP, BS), jnp.int32),
        scratch_shapes=[
            pltpu.VMEM((SEG, SEG), jnp.bfloat16),
            pltpu.VMEM((L, L), jnp.bfloat16),
        ],
    )(scores_t, thr, quo)


def _select_body(p_hbm, s_hbm, xf_hbm, idx_hbm, w_hbm, y_hbm,
                 lo_a, hi_a, gi_a, pv_a, lo_b, hi_b, gi_b, pv_b,
                 w_f, rows_v, sem_a, sem_b, sem_r):
    wid = lax.axis_index("s") * 2 + lax.axis_index("c")
    lane = lax.iota(jnp.int32, L)
    ea = wid * 2
    eb = ea + 1

    def init(lo, hi):
        def st(c, x):
            lo[pl.ds(c * L, L)] = jnp.zeros((L,), jnp.int32)
            hi[pl.ds(c * L, L)] = jnp.full((L,), BS - 1, jnp.int32)
            return x
        lax.fori_loop(0, K // L, st, 0)

    init(lo_a, hi_a)
    init(lo_b, hi_b)

    def mids(lo, hi, gi, base):
        def st(c, x):
            l = lo[pl.ds(c * L, L)]
            h = hi[pl.ds(c * L, L)]
            gi[pl.ds(c * L, L)] = ((l + h) >> 1) + base
            return x
        lax.fori_loop(0, K // L, st, 0)

    def update(lo, hi, gi, pv, base):
        def st(c, x):
            l = lo[pl.ds(c * L, L)]
            h = hi[pl.ds(c * L, L)]
            mid = gi[pl.ds(c * L, L)] - base
            v = pv[pl.ds(c * L, L)]
            tgt = lane + (c * L + 1)
            cond = v >= tgt
            hi[pl.ds(c * L, L)] = jnp.where(cond, mid, h)
            lo[pl.ds(c * L, L)] = jnp.where(cond, l, mid + 1)
            return x
        lax.fori_loop(0, K // L, st, 0)

    def round_(r, x):
        mids(lo_a, hi_a, gi_a, ea * BS)
        mids(lo_b, hi_b, gi_b, eb * BS)
        cp_a = pltpu.async_copy(p_hbm.at[gi_a], pv_a, sem_a)
        cp_b = pltpu.async_copy(p_hbm.at[gi_b], pv_b, sem_b)
        cp_a.wait()
        cp_b.wait()
        update(lo_a, hi_a, gi_a, pv_a, ea * BS)
        update(lo_b, hi_b, gi_b, pv_b, eb * BS)
        return x

    lax.fori_loop(0, 15, round_, 0)

    def finish(e, lo, gi, pv, sem):
        def st(c, x):
            gi[pl.ds(c * L, L)] = lo[pl.ds(c * L, L)] + e * BS
            return x
        lax.fori_loop(0, K // L, st, 0)
        pltpu.async_copy(s_hbm.at[gi], w_f, sem).wait()
        pltpu.sync_copy(lo, idx_hbm.at[e])
        pltpu.sync_copy(w_f, w_hbm.at[e])
        pltpu.async_copy(xf_hbm.at[lo], rows_v, sem_r).wait()
        pltpu.sync_copy(rows_v, y_hbm.at[pl.ds(e * K, K)])

    finish(ea, lo_a, gi_a, pv_a, sem_a)
    finish(eb, lo_b, gi_b, pv_b, sem_b)


def _select_and_gather(p_flat, s_flat, xf):
    mesh = plsc.VectorSubcoreMesh(core_axis_name="c", subcore_axis_name="s",
                                  num_cores=2, num_subcores=16)
    f = pl.kernel(
        _select_body,
        out_type=[
            jax.ShapeDtypeStruct((N_EXP, K), jnp.int32),
            jax.ShapeDtypeStruct((N_EXP, K), jnp.float32),
            jax.ShapeDtypeStruct((N_EXP * K, D), jnp.float32),
        ],
        mesh=mesh,
        scratch_types=(
            [pltpu.VMEM((K,), jnp.int32) for _ in range(8)]
            + [pltpu.VMEM((K,), jnp.float32),
               pltpu.VMEM((K, D), jnp.float32),
               pltpu.SemaphoreType.DMA,
               pltpu.SemaphoreType.DMA,
               pltpu.SemaphoreType.DMA]
        ),
    )
    return f(p_flat, s_flat, xf)


def _mlp_body(y_ref, w_ref, fc1_ref, fc2_ref, b1_ref, b2_ref, g_ref, be_ref,
              out_ref):
    y = y_ref[...]
    mu = jnp.mean(y, axis=1, keepdims=True)
    yc = y - mu
    var = jnp.mean(yc * yc, axis=1, keepdims=True)
    yn = yc * lax.rsqrt(var + LN_EPS) * g_ref[...] + be_ref[...]
    h = lax.dot_general(
        yn, fc1_ref[0], dimension_numbers=(((1,), (1,)), ((), ())),
        preferred_element_type=jnp.float32,
    ) + b1_ref[0]
    h = jax.nn.gelu(h, approximate=True)
    o = lax.dot_general(
        h, fc2_ref[0], dimension_numbers=(((1,), (1,)), ((), ())),
        preferred_element_type=jnp.float32,
    ) + b2_ref[0]
    out_ref[...] = o * w_ref[0].reshape(K, 1)


def _expert_mlp(y, w_t, fc1s, fc2s, b1s, b2s, gamma, beta, e0, ne):
    return pl.pallas_call(
        _mlp_body,
        grid=(ne,),
        in_specs=[
            pl.BlockSpec((K, D), lambda e: (e0 + e, 0)),
            pl.BlockSpec((1, 1, K), lambda e: (e0 + e, 0, 0)),
            pl.BlockSpec((1, DD, D), lambda e: (e0 + e, 0, 0)),
            pl.BlockSpec((1, D, DD), lambda e: (e0 + e, 0, 0)),
            pl.BlockSpec((1, 1, DD), lambda e: (e0 + e, 0, 0)),
            pl.BlockSpec((1, 1, D), lambda e: (e0 + e, 0, 0)),
            pl.BlockSpec((1, D), lambda e: (0, 0)),
            pl.BlockSpec((1, D), lambda e: (0, 0)),
        ],
        out_specs=pl.BlockSpec((K, D), lambda e: (e, 0)),
        out_shape=jax.ShapeDtypeStruct((ne * K, D), jnp.float32),
    )(y, w_t.reshape(N_EXP, 1, K), fc1s, fc2s,
      b1s.reshape(N_EXP, 1, DD), b2s.reshape(N_EXP, 1, D),
      gamma.reshape(1, D), beta.reshape(1, D))


def kernel(x, Wg, fc1s, fc2s, b1s, b2s, gamma, beta):
    og_shape = x.shape
    xf = x.reshape(-1, D)

    scores_t, xcopy = _gate_scores_t(xf, Wg)
    thr, quo = _thresholds(scores_t)
    p = _sel_prefix(scores_t, thr, quo)
    idx_t, w_t, y = _select_and_gather(
        p.reshape(-1), scores_t.reshape(-1), xf)
    nsplit = 8
    step = N_EXP // nsplit
    out = xcopy
    for i in range(nsplit):
        o_i = _expert_mlp(y, w_t, fc1s, fc2s, b1s, b2s, gamma, beta,
                          i * step, step)
        out = out.at[idx_t[i * step:(i + 1) * step].reshape(-1)].add(o_i)
    return out.reshape(og_shape)


# speedup vs baseline: 1.2565x; 1.2565x over previous
import functools

import jax
import jax.numpy as jnp
from jax import lax
from jax.experimental import pallas as pl
from jax.experimental.pallas import tpu as pltpu
import jax.experimental.pallas.tpu_sc as plsc

D = 768
DD = 3072
N_EXP = 64
K = 128
BS = 32768
LN_EPS = 1e-5

TOK_BLK = 2048
L = 16
SEG = 2048
NSEG = BS // SEG
EBLK = 8


def _gate_body(x_ref, wg_ref, out_ref, xc_ref):
    logits = lax.dot_general(
        wg_ref[...], x_ref[...],
        dimension_numbers=(((1,), (1,)), ((), ())),
        preferred_element_type=jnp.float32,
    )
    m = jnp.max(logits, axis=0, keepdims=True)
    e = jnp.exp(logits - m)
    out_ref[...] = e / jnp.sum(e, axis=0, keepdims=True)
    xc_ref[...] = x_ref[...]


def _gate_scores_t(xf, Wg):
    return pl.pallas_call(
        _gate_body,
        grid=(BS // TOK_BLK,),
        in_specs=[
            pl.BlockSpec((TOK_BLK, D), lambda i: (i, 0)),
            pl.BlockSpec((N_EXP, D), lambda i: (0, 0)),
        ],
        out_specs=[
            pl.BlockSpec((N_EXP, TOK_BLK), lambda i: (0, i)),
            pl.BlockSpec((TOK_BLK, D), lambda i: (i, 0)),
        ],
        out_shape=[
            jax.ShapeDtypeStruct((N_EXP, BS), jnp.float32),
            jax.ShapeDtypeStruct((BS, D), jnp.float32),
        ],
    )(xf, Wg)


def _thr_body(s_ref, thr_ref, quo_ref):
    bits = lax.bitcast_convert_type(s_ref[...], jnp.int32)

    def step(_, lohi):
        lo, hi = lohi
        mid = (lo + hi + 1) >> 1
        cnt = jnp.sum((bits >= mid).astype(jnp.int32), axis=1, keepdims=True)
        ge = cnt >= K
        return jnp.where(ge, mid, lo), jnp.where(ge, hi, mid - 1)

    lo0 = jnp.zeros((N_EXP, 1), jnp.int32)
    hi0 = jnp.full((N_EXP, 1), 0x3F800000, jnp.int32)
    lo, _ = lax.fori_loop(0, 31, step, (lo0, hi0))
    cnt_gt = jnp.sum((bits > lo).astype(jnp.int32), axis=1, keepdims=True)
    thr_ref[...] = jnp.broadcast_to(
        lax.bitcast_convert_type(lo, jnp.float32), (N_EXP, L))
    quo_ref[...] = jnp.broadcast_to(K - cnt_gt, (N_EXP, L))


def _thresholds(scores_t):
    return pl.pallas_call(
        _thr_body,
        in_specs=[pl.BlockSpec((N_EXP, BS), lambda: (0, 0))],
        out_specs=[
            pl.BlockSpec((N_EXP, L), lambda: (0, 0)),
            pl.BlockSpec((N_EXP, L), lambda: (0, 0)),
        ],
        out_shape=[
            jax.ShapeDtypeStruct((N_EXP, L), jnp.float32),
            jax.ShapeDtypeStruct((N_EXP, L), jnp.int32),
        ],
    )(scores_t)


def _prefix_body(s_ref, thr_ref, quo_ref, p_ref, tri_ref, tri16_ref):
    @pl.when(pl.program_id(0) == 0)
    def _():
        r = lax.broadcasted_iota(jnp.int32, (SEG, SEG), 0)
        c = lax.broadcasted_iota(jnp.int32, (SEG, SEG), 1)
        tri_ref[...] = jnp.where(r <= c, 1.0, 0.0).astype(jnp.bfloat16)
        r16 = lax.broadcasted_iota(jnp.int32, (L, L), 0)
        c16 = lax.broadcasted_iota(jnp.int32, (L, L), 1)
        tri16_ref[...] = jnp.where(r16 < c16, 1.0, 0.0).astype(jnp.bfloat16)

    s = s_ref[...]
    thr = thr_ref[:, :1]
    quo = quo_ref[:, :1].astype(jnp.float32)

    def prefix(mask_f):
        m = mask_f.reshape(EBLK * NSEG, SEG).astype(jnp.bfloat16)
        pref = lax.dot_general(
            m, tri_ref[...], dimension_numbers=(((1,), (0,)), ((), ())),
            preferred_element_type=jnp.float32,
        )
        seg_tot = pref[:, SEG - 1].reshape(EBLK, NSEG).astype(jnp.bfloat16)
        seg_off = lax.dot_general(
            seg_tot, tri16_ref[...], dimension_numbers=(((1,), (0,)), ((), ())),
            preferred_element_type=jnp.float32,
        )
        return (pref.reshape(EBLK, NSEG, SEG)
                + seg_off[:, :, None]).reshape(EBLK, BS)

    p_gt = prefix(jnp.where(s > thr, 1.0, 0.0))
    p_eq = prefix(jnp.where(s == thr, 1.0, 0.0))
    p_ref[...] = (p_gt + jnp.minimum(p_eq, quo)).astype(jnp.int32)


def _sel_prefix(scores_t, thr, quo):
    return pl.pallas_call(
        _prefix_body,
        grid=(N_EXP // EBLK,),
        in_specs=[
            pl.BlockSpec((EBLK, BS), lambda i: (i, 0)),
            pl.BlockSpec((EBLK, L), lambda i: (i, 0)),
            pl.BlockSpec((EBLK, L), lambda i: (i, 0)),
        ],
        out_specs=pl.BlockSpec((EBLK, BS), lambda i: (i, 0)),
        out_shape=jax.ShapeDtypeStruct((N_EXP, BS), jnp.int32),
        scratch_shapes=[
            pltpu.VMEM((SEG, SEG), jnp.bfloat16),
            pltpu.VMEM((L, L), jnp.bfloat16),
        ],
    )(scores_t, thr, quo)


def _select_body(p_hbm, s_hbm, xf_hbm, idx_hbm, w_hbm, y_hbm,
                 lo_a, hi_a, gi_a, pv_a, lo_b, hi_b, gi_b, pv_b,
                 w_f, rows_v, sem_a, sem_b, sem_r):
    wid = lax.axis_index("s") * 2 + lax.axis_index("c")
    lane = lax.iota(jnp.int32, L)
    ea = wid * 2
    eb = ea + 1

    def init(lo, hi):
        def st(c, x):
            lo[pl.ds(c * L, L)] = jnp.zeros((L,), jnp.int32)
            hi[pl.ds(c * L, L)] = jnp.full((L,), BS - 1, jnp.int32)
            return x
        lax.fori_loop(0, K // L, st, 0)

    init(lo_a, hi_a)
    init(lo_b, hi_b)

    def mids(lo, hi, gi, base):
        def st(c, x):
            l = lo[pl.ds(c * L, L)]
            h = hi[pl.ds(c * L, L)]
            gi[pl.ds(c * L, L)] = ((l + h) >> 1) + base
            return x
        lax.fori_loop(0, K // L, st, 0)

    def update(lo, hi, gi, pv, base):
        def st(c, x):
            l = lo[pl.ds(c * L, L)]
            h = hi[pl.ds(c * L, L)]
            mid = gi[pl.ds(c * L, L)] - base
            v = pv[pl.ds(c * L, L)]
            tgt = lane + (c * L + 1)
            cond = v >= tgt
            hi[pl.ds(c * L, L)] = jnp.where(cond, mid, h)
            lo[pl.ds(c * L, L)] = jnp.where(cond, l, mid + 1)
            return x
        lax.fori_loop(0, K // L, st, 0)

    def round_(r, x):
        mids(lo_a, hi_a, gi_a, ea * BS)
        mids(lo_b, hi_b, gi_b, eb * BS)
        cp_a = pltpu.async_copy(p_hbm.at[gi_a], pv_a, sem_a)
        cp_b = pltpu.async_copy(p_hbm.at[gi_b], pv_b, sem_b)
        cp_a.wait()
        cp_b.wait()
        update(lo_a, hi_a, gi_a, pv_a, ea * BS)
        update(lo_b, hi_b, gi_b, pv_b, eb * BS)
        return x

    lax.fori_loop(0, 15, round_, 0)

    def finish(e, lo, gi, pv, sem):
        def st(c, x):
            gi[pl.ds(c * L, L)] = lo[pl.ds(c * L, L)] + e * BS
            return x
        lax.fori_loop(0, K // L, st, 0)
        pltpu.async_copy(s_hbm.at[gi], w_f, sem).wait()
        pltpu.sync_copy(lo, idx_hbm.at[e])
        pltpu.sync_copy(w_f, w_hbm.at[e])
        pltpu.async_copy(xf_hbm.at[lo], rows_v, sem_r).wait()
        pltpu.sync_copy(rows_v, y_hbm.at[pl.ds(e * K, K)])

    finish(ea, lo_a, gi_a, pv_a, sem_a)
    finish(eb, lo_b, gi_b, pv_b, sem_b)


def _select_and_gather(p_flat, s_flat, xf):
    mesh = plsc.VectorSubcoreMesh(core_axis_name="c", subcore_axis_name="s",
                                  num_cores=2, num_subcores=16)
    f = pl.kernel(
        _select_body,
        out_type=[
            jax.ShapeDtypeStruct((N_EXP, K), jnp.int32),
            jax.ShapeDtypeStruct((N_EXP, K), jnp.float32),
            jax.ShapeDtypeStruct((N_EXP * K, D), jnp.float32),
        ],
        mesh=mesh,
        scratch_types=(
            [pltpu.VMEM((K,), jnp.int32) for _ in range(8)]
            + [pltpu.VMEM((K,), jnp.float32),
               pltpu.VMEM((K, D), jnp.float32),
               pltpu.SemaphoreType.DMA,
               pltpu.SemaphoreType.DMA,
               pltpu.SemaphoreType.DMA]
        ),
    )
    return f(p_flat, s_flat, xf)


def _mlp_body(y_ref, w_ref, fc1_ref, fc2_ref, b1_ref, b2_ref, g_ref, be_ref,
              out_ref):
    y = y_ref[...]
    mu = jnp.mean(y, axis=1, keepdims=True)
    yc = y - mu
    var = jnp.mean(yc * yc, axis=1, keepdims=True)
    yn = yc * lax.rsqrt(var + LN_EPS) * g_ref[...] + be_ref[...]
    h = lax.dot_general(
        yn, fc1_ref[0], dimension_numbers=(((1,), (1,)), ((), ())),
        preferred_element_type=jnp.float32,
    ) + b1_ref[0]
    h = jax.nn.gelu(h, approximate=True)
    o = lax.dot_general(
        h, fc2_ref[0], dimension_numbers=(((1,), (1,)), ((), ())),
        preferred_element_type=jnp.float32,
    ) + b2_ref[0]
    out_ref[...] = o * w_ref[0].reshape(K, 1)


def _expert_mlp(y, w_t, fc1s, fc2s, b1s, b2s, gamma, beta, e0, ne):
    return pl.pallas_call(
        _mlp_body,
        grid=(ne,),
        in_specs=[
            pl.BlockSpec((K, D), lambda e: (e0 + e, 0)),
            pl.BlockSpec((1, 1, K), lambda e: (e0 + e, 0, 0)),
            pl.BlockSpec((1, DD, D), lambda e: (e0 + e, 0, 0)),
            pl.BlockSpec((1, D, DD), lambda e: (e0 + e, 0, 0)),
            pl.BlockSpec((1, 1, DD), lambda e: (e0 + e, 0, 0)),
            pl.BlockSpec((1, 1, D), lambda e: (e0 + e, 0, 0)),
            pl.BlockSpec((1, D), lambda e: (0, 0)),
            pl.BlockSpec((1, D), lambda e: (0, 0)),
        ],
        out_specs=pl.BlockSpec((K, D), lambda e: (e, 0)),
        out_shape=jax.ShapeDtypeStruct((ne * K, D), jnp.float32),
    )(y, w_t.reshape(N_EXP, 1, K), fc1s, fc2s,
      b1s.reshape(N_EXP, 1, DD), b2s.reshape(N_EXP, 1, D),
      gamma.reshape(1, D), beta.reshape(1, D))


def kernel(x, Wg, fc1s, fc2s, b1s, b2s, gamma, beta):
    og_shape = x.shape
    xf = x.reshape(-1, D)

    scores_t, xcopy = _gate_scores_t(xf, Wg)
    thr, quo = _thresholds(scores_t)
    p = _sel_prefix(scores_t, thr, quo)
    idx_t, w_t, y = _select_and_gather(
        p.reshape(-1), scores_t.reshape(-1), xf)
    nsplit = 4
    step = N_EXP // nsplit
    out = xcopy
    for i in range(nsplit):
        o_i = _expert_mlp(y, w_t, fc1s, fc2s, b1s, b2s, gamma, beta,
                          i * step, step)
        out = out.at[idx_t[i * step:(i + 1) * step].reshape(-1)].add(o_i)
    return out.reshape(og_shape)
